# baseline (speedup 1.0000x reference)
import jax
import jax.numpy as jnp
from jax.experimental import pallas as pl
from jax.experimental.pallas import tpu as pltpu

B, C, H, W = 32, 384, 14, 14
N = H * W
K = 9
OUT = 384
RED = 16

_F32 = jnp.float32
_BF16 = jnp.bfloat16


def _dgc_kernel(x_ref, wcat_ref, b_ref, fc1_ref, fc2_ref, tband_ref, sab_ref,
                out_ref):
    x = x_ref[0]

    avg = jnp.mean(x, axis=1, keepdims=True)
    mx = jnp.max(x, axis=1, keepdims=True)
    fc1 = fc1_ref[...]
    fc2 = fc2_ref[...]

    def mlp(v):
        h = jnp.maximum(jnp.dot(fc1, v, preferred_element_type=_F32), 0.0)
        return jnp.dot(fc2, h, preferred_element_type=_F32)

    ca = jax.nn.sigmoid(mlp(avg) + mlp(mx))
    x1 = x * ca

    m_mean = jnp.mean(x1, axis=0, keepdims=True)
    m_max = jnp.max(x1, axis=0, keepdims=True)
    sa_in = jnp.concatenate([m_mean, m_max], axis=1)
    sa = jnp.dot(sa_in, tband_ref[...],
                 preferred_element_type=_F32) + sab_ref[...]
    att = x1 * jax.nn.sigmoid(sa)

    amax = jnp.max(att, axis=1, keepdims=True)
    e = jnp.exp(att - amax)
    soft = e / jnp.sum(e, axis=1, keepdims=True)
    smax = jnp.max(soft, axis=1, keepdims=True)
    att = soft / (smax + 1e-10)
    att = (2.0 * att - 1.0) / 40.0 + 1.0
    xa = x * att

    nrm = jnp.sqrt(jnp.sum(x * x, axis=0, keepdims=True))
    v = x / (nrm + 1e-12)
    vT = v.T
    sq_col = jnp.sum(vT * vT, axis=1, keepdims=True)
    g = jnp.dot(vT, v, preferred_element_type=_F32)
    neg = 2.0 * g - sq_col - sq_col.T

    hits = []
    for k in range(K):
        mval = jnp.max(neg, axis=1, keepdims=True)
        hit = neg == mval
        hits.append(hit.astype(_BF16))
        if k < K - 1:
            neg = jnp.where(hit, -jnp.inf, neg)

    ab = jnp.dot(wcat_ref[...].astype(_BF16), xa.astype(_BF16),
                 preferred_element_type=_F32)
    a_part = jax.lax.slice(ab, (0, 0), (OUT, N))
    b_part = jax.lax.slice(ab, (OUT, 0), (2 * OUT, N))

    b_hi16 = b_part.astype(_BF16)
    b_lo16 = (b_part - b_hi16.astype(_F32)).astype(_BF16)
    dn = (((1,), (1,)), ((), ()))
    m = jnp.full((OUT, N), -jnp.inf, _F32)
    for k in range(K):
        gk = (jax.lax.dot_general(b_hi16, hits[k], dn,
                                  preferred_element_type=_F32)
              + jax.lax.dot_general(b_lo16, hits[k], dn,
                                    preferred_element_type=_F32))
        m = jnp.maximum(m, gk)

    out_ref[0] = jnp.maximum(a_part + m + b_ref[...], 0.0)


def _conv_band_matrix(sa_w):
    w = sa_w.reshape(2, 7, 7)
    pos = jnp.arange(N)
    py, px = pos // W, pos % W
    dy = py[:, None] - py[None, :] + 3
    dx = px[:, None] - px[None, :] + 3
    valid = (dy >= 0) & (dy <= 6) & (dx >= 0) & (dx <= 6)
    dyc = jnp.clip(dy, 0, 6)
    dxc = jnp.clip(dx, 0, 6)
    t = jnp.where(valid[None], w[:, dyc, dxc], 0.0)
    return t.reshape(2 * N, N)


def kernel(x, conv_w, conv_b, ca_fc1, ca_fc2, sa_w, sa_b):
    xr = x.reshape(B, C, N)
    w1 = conv_w[:, :C]
    w2 = conv_w[:, C:]
    wcat = jnp.concatenate([w1 - w2, w2], axis=0)
    bcol = conv_b.reshape(OUT, 1)
    tband = _conv_band_matrix(sa_w)
    sab = sa_b.reshape(1, 1)

    out = pl.pallas_call(
        _dgc_kernel,
        grid=(B,),
        in_specs=[
            pl.BlockSpec((1, C, N), lambda i: (i, 0, 0)),
            pl.BlockSpec((2 * OUT, C), lambda i: (0, 0)),
            pl.BlockSpec((OUT, 1), lambda i: (0, 0)),
            pl.BlockSpec((C // RED, C), lambda i: (0, 0)),
            pl.BlockSpec((C, C // RED), lambda i: (0, 0)),
            pl.BlockSpec((2 * N, N), lambda i: (0, 0)),
            pl.BlockSpec((1, 1), lambda i: (0, 0)),
        ],
        out_specs=pl.BlockSpec((1, OUT, N), lambda i: (i, 0, 0)),
        out_shape=jax.ShapeDtypeStruct((B, OUT, N), _F32),
    )(xr, wcat, bcol, ca_fc1, ca_fc2, tband, sab)
    return out.reshape(B, OUT, H, W)


# speedup vs baseline: 2.7120x; 2.7120x over previous
import jax
import jax.numpy as jnp
from jax.experimental import pallas as pl
from jax.experimental.pallas import tpu as pltpu

B, C, H, W = 32, 384, 14, 14
N = H * W
K = 9
OUT = 384
RED = 16

_F32 = jnp.float32
_HI = jax.lax.Precision.HIGHEST


def _dgc_kernel(x_ref, wcat_ref, b_ref, fc1_ref, fc2_ref, saw_ref, sab_ref,
                out_ref):
    x = x_ref[0]

    avg = jnp.mean(x, axis=1, keepdims=True)
    mx = jnp.max(x, axis=1, keepdims=True)
    fc1 = fc1_ref[...]
    fc2 = fc2_ref[...]

    def mlp(v):
        h = jnp.maximum(jnp.dot(fc1, v, preferred_element_type=_F32), 0.0)
        return jnp.dot(fc2, h, preferred_element_type=_F32)

    ca = jax.nn.sigmoid(mlp(avg) + mlp(mx))
    x1 = x * ca

    m_mean = jnp.mean(x1, axis=0, keepdims=True)
    m_max = jnp.max(x1, axis=0, keepdims=True)
    sa_in = jnp.concatenate([m_mean, m_max], axis=0)
    PAD = 45
    padded = jnp.concatenate(
        [jnp.zeros((2, PAD), _F32), sa_in, jnp.zeros((2, PAD), _F32)], axis=1)
    px = jax.lax.broadcasted_iota(jnp.int32, (1, N), 1) % W
    saw = saw_ref[...]
    acc = jnp.zeros((1, N), _F32)
    for dy in range(-3, 4):
        for dx in range(-3, 4):
            s = dy * W + dx
            sh = jax.lax.slice(padded, (0, PAD + s), (2, PAD + s + N))
            wcol = jax.lax.slice(saw, (0, (dy + 3) * 7 + (dx + 3)),
                                 (2, (dy + 3) * 7 + (dx + 3) + 1))
            msk = ((px + dx >= 0) & (px + dx < W)).astype(_F32)
            acc = acc + jnp.sum(sh * wcol, axis=0, keepdims=True) * msk
    sa = acc + sab_ref[...]
    att = x1 * jax.nn.sigmoid(sa)

    amax = jnp.max(att, axis=1, keepdims=True)
    e = jnp.exp(att - amax)
    soft = e / jnp.sum(e, axis=1, keepdims=True)
    smax = jnp.max(soft, axis=1, keepdims=True)
    att = soft / (smax + 1e-10)
    att = (2.0 * att - 1.0) / 40.0 + 1.0
    xa = x * att

    nrm = jnp.sqrt(jnp.sum(x * x, axis=0, keepdims=True))
    v = x / (nrm + 1e-12)
    vT = v.T
    sq_col = jnp.sum(vT * vT, axis=1, keepdims=True)
    g = jnp.dot(vT, v, preferred_element_type=_F32)
    neg = 2.0 * g - sq_col - sq_col.T

    iota_l = jax.lax.broadcasted_iota(jnp.int32, (N, N), 1)
    idx_rows = []
    for _ in range(K):
        mval = jnp.max(neg, axis=1, keepdims=True)
        cand = jnp.where(neg == mval, iota_l, N)
        idxk = jnp.min(cand, axis=1, keepdims=True)
        idx_rows.append(idxk.T)
        neg = jnp.where(iota_l == idxk, -jnp.inf, neg)

    ab = jnp.dot(wcat_ref[...].astype(jnp.bfloat16), xa.astype(jnp.bfloat16),
                 preferred_element_type=_F32)
    a_part = jax.lax.slice(ab, (0, 0), (OUT, N))
    b_part = jax.lax.slice(ab, (OUT, 0), (2 * OUT, N))

    b_hi16 = b_part.astype(jnp.bfloat16)
    b_lo16 = (b_part - b_hi16.astype(_F32)).astype(jnp.bfloat16)
    iota_s = jax.lax.broadcasted_iota(jnp.int32, (N, N), 0)
    m = jnp.full((OUT, N), -jnp.inf, _F32)
    for k in range(K):
        p = (iota_s == idx_rows[k]).astype(jnp.bfloat16)
        gk = (jnp.dot(b_hi16, p, preferred_element_type=_F32)
              + jnp.dot(b_lo16, p, preferred_element_type=_F32))
        m = jnp.maximum(m, gk)

    out_ref[0] = jnp.maximum(a_part + m + b_ref[...], 0.0)


def kernel(x, conv_w, conv_b, ca_fc1, ca_fc2, sa_w, sa_b):
    xr = x.reshape(B, C, N)
    w1 = conv_w[:, :C]
    w2 = conv_w[:, C:]
    wcat = jnp.concatenate([w1 - w2, w2], axis=0)
    bcol = conv_b.reshape(OUT, 1)
    saw = sa_w.reshape(2, 49)
    sab = sa_b.reshape(1, 1)

    out = pl.pallas_call(
        _dgc_kernel,
        grid=(B,),
        in_specs=[
            pl.BlockSpec((1, C, N), lambda i: (i, 0, 0)),
            pl.BlockSpec((2 * OUT, C), lambda i: (0, 0)),
            pl.BlockSpec((OUT, 1), lambda i: (0, 0)),
            pl.BlockSpec((C // RED, C), lambda i: (0, 0)),
            pl.BlockSpec((C, C // RED), lambda i: (0, 0)),
            pl.BlockSpec((2, 49), lambda i: (0, 0)),
            pl.BlockSpec((1, 1), lambda i: (0, 0)),
        ],
        out_specs=pl.BlockSpec((1, OUT, N), lambda i: (i, 0, 0)),
        out_shape=jax.ShapeDtypeStruct((B, OUT, N), _F32),
    )(xr, wcat, bcol, ca_fc1, ca_fc2, saw, sab)
    return out.reshape(B, OUT, H, W)


# speedup vs baseline: 3.6363x; 1.3408x over previous
import jax
import jax.numpy as jnp
from jax.experimental import pallas as pl
from jax.experimental.pallas import tpu as pltpu

B, C, H, W = 32, 384, 14, 14
N = H * W
K = 9
OUT = 384
RED = 16

_F32 = jnp.float32
_HI = jax.lax.Precision.HIGHEST


def _dgc_kernel(x_ref, wcat_ref, b_ref, fc1_ref, fc2_ref, saw_ref, sab_ref,
                out_ref):
    x = x_ref[0]

    avg = jnp.mean(x, axis=1, keepdims=True)
    mx = jnp.max(x, axis=1, keepdims=True)
    fc1 = fc1_ref[...]
    fc2 = fc2_ref[...]

    def mlp(v):
        h = jnp.maximum(jnp.dot(fc1, v, preferred_element_type=_F32), 0.0)
        return jnp.dot(fc2, h, preferred_element_type=_F32)

    ca = jax.nn.sigmoid(mlp(avg) + mlp(mx))
    x1 = x * ca

    m_mean = jnp.mean(x1, axis=0, keepdims=True)
    m_max = jnp.max(x1, axis=0, keepdims=True)
    sa_in = jnp.concatenate([m_mean, m_max], axis=0)
    PAD = 45
    padded = jnp.concatenate(
        [jnp.zeros((2, PAD), _F32), sa_in, jnp.zeros((2, PAD), _F32)], axis=1)
    px = jax.lax.broadcasted_iota(jnp.int32, (1, N), 1) % W
    saw = saw_ref[...]
    acc = jnp.zeros((1, N), _F32)
    for dy in range(-3, 4):
        for dx in range(-3, 4):
            s = dy * W + dx
            sh = jax.lax.slice(padded, (0, PAD + s), (2, PAD + s + N))
            wcol = jax.lax.slice(saw, (0, (dy + 3) * 7 + (dx + 3)),
                                 (2, (dy + 3) * 7 + (dx + 3) + 1))
            msk = ((px + dx >= 0) & (px + dx < W)).astype(_F32)
            acc = acc + jnp.sum(sh * wcol, axis=0, keepdims=True) * msk
    sa = acc + sab_ref[...]
    att = x1 * jax.nn.sigmoid(sa)

    amax = jnp.max(att, axis=1, keepdims=True)
    e = jnp.exp(att - amax)
    soft = e / jnp.sum(e, axis=1, keepdims=True)
    smax = jnp.max(soft, axis=1, keepdims=True)
    att = soft / (smax + 1e-10)
    att = (2.0 * att - 1.0) / 40.0 + 1.0
    xa = x * att

    nrm = jnp.sqrt(jnp.sum(x * x, axis=0, keepdims=True))
    v = x / (nrm + 1e-12)
    vT = v.T
    sq_col = jnp.sum(vT * vT, axis=1, keepdims=True)
    g = jnp.dot(vT, v, preferred_element_type=_F32)
    neg = 2.0 * g - sq_col - sq_col.T

    hit_masks = []
    for k in range(K):
        mval = jnp.max(neg, axis=0, keepdims=True)
        hit = neg == mval
        hit_masks.append(hit.astype(jnp.bfloat16))
        if k < K - 1:
            neg = jnp.where(hit, -jnp.inf, neg)

    ab = jnp.dot(wcat_ref[...].astype(jnp.bfloat16), xa.astype(jnp.bfloat16),
                 preferred_element_type=_F32)
    a_part = jax.lax.slice(ab, (0, 0), (OUT, N))
    b_part = jax.lax.slice(ab, (OUT, 0), (2 * OUT, N))

    b_hi16 = b_part.astype(jnp.bfloat16)
    b_lo16 = (b_part - b_hi16.astype(_F32)).astype(jnp.bfloat16)
    m = jnp.full((OUT, N), -jnp.inf, _F32)
    for k in range(K):
        p = hit_masks[k]
        gk = (jnp.dot(b_hi16, p, preferred_element_type=_F32)
              + jnp.dot(b_lo16, p, preferred_element_type=_F32))
        m = jnp.maximum(m, gk)

    out_ref[0] = jnp.maximum(a_part + m + b_ref[...], 0.0)


def kernel(x, conv_w, conv_b, ca_fc1, ca_fc2, sa_w, sa_b):
    xr = x.reshape(B, C, N)
    w1 = conv_w[:, :C]
    w2 = conv_w[:, C:]
    wcat = jnp.concatenate([w1 - w2, w2], axis=0)
    bcol = conv_b.reshape(OUT, 1)
    saw = sa_w.reshape(2, 49)
    sab = sa_b.reshape(1, 1)

    out = pl.pallas_call(
        _dgc_kernel,
        grid=(B,),
        in_specs=[
            pl.BlockSpec((1, C, N), lambda i: (i, 0, 0)),
            pl.BlockSpec((2 * OUT, C), lambda i: (0, 0)),
            pl.BlockSpec((OUT, 1), lambda i: (0, 0)),
            pl.BlockSpec((C // RED, C), lambda i: (0, 0)),
            pl.BlockSpec((C, C // RED), lambda i: (0, 0)),
            pl.BlockSpec((2, 49), lambda i: (0, 0)),
            pl.BlockSpec((1, 1), lambda i: (0, 0)),
        ],
        out_specs=pl.BlockSpec((1, OUT, N), lambda i: (i, 0, 0)),
        out_shape=jax.ShapeDtypeStruct((B, OUT, N), _F32),
    )(xr, wcat, bcol, ca_fc1, ca_fc2, saw, sab)
    return out.reshape(B, OUT, H, W)
